# baseline (speedup 1.0000x reference)
import jax
import jax.numpy as jnp
from jax import lax
from jax.experimental import pallas as pl
from jax.experimental.pallas import tpu as pltpu
from jax.experimental.pallas import tpu_sc as plsc

N = 10000
E = 320000
D = 128
DH = D // 2

NC = 2
NS = 16
L = 16

N_PAD = 10112
EPT = E // NS
CHUNK = 80
CPT = EPT // CHUNK
CPW = E // (NC * NS) // CHUNK
RPT = N_PAD // NS
ZROWS = 79
K = 5
NHALF0 = 5120
NHALF1 = N_PAD - NHALF0
NTRASH = 8
CROWS = NHALF0 // NS
CROWS1 = NHALF1 // NS

_MESH = dict(core_axis_name="c", subcore_axis_name="s",
             num_cores=NC, num_subcores=NS)
_PARAMS = pltpu.CompilerParams(use_tc_tiling_on_sc=False)


def _seg_sum_body(with_cnt, x2_hbm, src_hbm, dst_hbm, agg_out, *rest):
  if with_cnt:
    (cnt_out, idx_all, dst_all, rows, zbuf, agg_sh, sem_g, sem_s,
     ones_v, zcnt, cnt_sh, sem_c, cidx) = rest
  else:
    idx_all, dst_all, rows, zbuf, agg_sh, sem_g, sem_s = rest

  cid = lax.axis_index("c")
  sid = lax.axis_index("s")

  zero16 = jnp.zeros((L,), jnp.float32)
  ones16 = jnp.ones((L,), jnp.float32)

  pltpu.sync_copy(src_hbm.at[pl.ds(sid * CPT, CPT)], idx_all)
  pltpu.sync_copy(dst_hbm.at[pl.ds(sid * CPT, CPT)], dst_all)

  def ix(i, _):
    r = i // (CHUNK // L)
    k = i % (CHUNK // L)
    sv = idx_all[r, pl.ds(k * L, L)]
    idx_all[r, pl.ds(k * L, L)] = sv * 2 + cid
    return 0
  lax.fori_loop(0, CPT * (CHUNK // L), ix, 0)

  def zb(i, _):
    r = i // (DH // L)
    k = i % (DH // L)
    zbuf[r, pl.ds(k * L, L)] = zero16
    return 0
  lax.fori_loop(0, ZROWS * (DH // L), zb, 0)

  def zcpy(j, _):
    pltpu.sync_copy(zbuf, agg_sh.at[pl.ds(sid * RPT + j * ZROWS, ZROWS)])
    return 0
  lax.fori_loop(0, RPT // ZROWS, zcpy, 0)

  if with_cnt:
    def fo(i, _):
      ones_v[i, pl.ds(0, L)] = ones16
      return 0
    lax.fori_loop(0, CHUNK, fo, 0)

    def zc(i, _):
      zcnt[i, pl.ds(0, L)] = zero16
      return 0
    lax.fori_loop(0, CROWS, zc, 0)
    pltpu.sync_copy(zcnt.at[pl.ds(0, CROWS)],
                    cnt_sh.at[pl.ds(sid * CROWS, CROWS)])

    @pl.when(sid == 0)
    def _():
      pltpu.sync_copy(zcnt.at[pl.ds(0, NTRASH)],
                      cnt_sh.at[pl.ds(NS * CROWS, NTRASH)])

  plsc.subcore_barrier()

  def gather(c, k):
    return pltpu.make_async_copy(x2_hbm.at[idx_all.at[c]], rows[k], sem_g[k])

  def scatter(c, k):
    return pltpu.make_async_copy(rows[k], agg_sh.at[dst_all.at[c]], sem_s[k])

  def count(j):
    return pltpu.make_async_copy(ones_v, cnt_sh.at[cidx[j]], sem_c[j])

  if not with_cnt:
    for k in range(K - 1):
      gather(k, k).start()

    def step(q, _):
      for k in range(K):
        c = K * q + k
        gather(c, k).wait()
        pltpu.async_copy(rows[k], agg_sh.at[dst_all.at[c]], sem_s[k],
                         add=True)
        k3 = (k + K - 1) % K

        @pl.when(c + K - 1 < CPT)
        def _():
          @pl.when(c >= 1)
          def _():
            scatter(c - 1, k3).wait()
          gather(c + K - 1, k3).start()
      return 0

    lax.fori_loop(0, CPT // K, step, 0)
    for k in range(K):
      scatter(CPT - K + k, k).wait()
  else:
    U = 2 * K
    for k in range(K - 2):
      gather(k, k).start()

    def step(q, _):
      for k in range(U):
        c = U * q + k
        k5 = k % K
        j = k % 2
        gather(c, k5).wait()

        @pl.when(c >= 2)
        def _():
          scatter(c - 2, j).wait()
          count(j).wait()
        pltpu.async_copy(rows[k5], agg_sh.at[dst_all.at[c]], sem_s[j],
                         add=True)
        clim = NHALF0 - cid * (NHALF0 - NHALF1)

        def remap(j5, _):
          d = dst_all[c, pl.ds(j5 * L, L)]
          t = d - cid * NHALF0
          ok = (t >= 0) & (t < clim)
          cidx[j][pl.ds(j5 * L, L)] = jnp.where(ok, t, clim)
          return 0
        lax.fori_loop(0, CHUNK // L, remap, 0)
        pltpu.async_copy(ones_v, cnt_sh.at[cidx[j]], sem_c[j], add=True)
        kf = (k5 + K - 2) % K

        @pl.when(c + K - 2 < CPT)
        def _():
          gather(c + K - 2, kf).start()
      return 0

    lax.fori_loop(0, CPT // U, step, 0)
    for j in range(2):
      scatter(CPT - 2 + j, j).wait()
      count(j).wait()
  plsc.subcore_barrier()

  pltpu.sync_copy(agg_sh.at[pl.ds(sid * RPT, RPT)],
                  agg_out.at[cid, pl.ds(sid * RPT, RPT)])
  if with_cnt:
    @pl.when(cid == 0)
    def _():
      pltpu.sync_copy(cnt_sh.at[pl.ds(sid * CROWS, CROWS)],
                      cnt_out.at[pl.ds(sid * CROWS, CROWS)])

    @pl.when(cid == 1)
    def _():
      pltpu.sync_copy(cnt_sh.at[pl.ds(sid * CROWS1, CROWS1)],
                      cnt_out.at[pl.ds(NHALF0 + sid * CROWS1, CROWS1)])


def _make_seg_sum(with_cnt):
  out_type = [jax.ShapeDtypeStruct((NC, N_PAD, DH), jnp.float32)]
  scratch = [
      pltpu.VMEM((CPT, CHUNK), jnp.int32),
      pltpu.VMEM((CPT, CHUNK), jnp.int32),
      [pltpu.VMEM((CHUNK, DH), jnp.float32) for _ in range(K)],
      pltpu.VMEM((ZROWS, DH), jnp.float32),
      pltpu.VMEM_SHARED((N_PAD, DH), jnp.float32),
      [pltpu.SemaphoreType.DMA for _ in range(K)],
      [pltpu.SemaphoreType.DMA for _ in range(2 if with_cnt else K)],
  ]
  if with_cnt:
    out_type.append(jax.ShapeDtypeStruct((N_PAD, L), jnp.float32))
    scratch += [
        pltpu.VMEM((CHUNK, L), jnp.float32),
        pltpu.VMEM((CROWS, L), jnp.float32),
        pltpu.VMEM_SHARED((NHALF0 + NTRASH, L), jnp.float32),
        [pltpu.SemaphoreType.DMA for _ in range(2)],
        [pltpu.VMEM((CHUNK,), jnp.int32) for _ in range(2)],
    ]
  import functools
  return pl.kernel(
      functools.partial(_seg_sum_body, with_cnt),
      out_type=tuple(out_type) if with_cnt else out_type[0],
      mesh=plsc.VectorSubcoreMesh(**_MESH),
      scratch_types=tuple(scratch),
      compiler_params=_PARAMS)


_seg_sum_cnt = _make_seg_sum(True)
_seg_sum = _make_seg_sum(False)


def _dense(agg_p, cnt_p, xin, Wl, Wr, b, relu):
  BN = 512

  def body(aggp_ref, cnt_ref, x_ref, wl_ref, wr_ref, b_ref, o_ref):
    agg = jnp.concatenate([aggp_ref[0], aggp_ref[1]], axis=-1)
    cnt = cnt_ref[:, 0]
    inv = 1.0 / jnp.maximum(cnt, 1.0)
    mean = agg * inv[:, None]
    out = (jnp.dot(mean, wl_ref[...], preferred_element_type=jnp.float32)
           + jnp.dot(x_ref[...], wr_ref[...], preferred_element_type=jnp.float32)
           + b_ref[...])
    if relu:
      out = jnp.maximum(out, 0.0)
    o_ref[...] = out

  return pl.pallas_call(
      body,
      grid=(pl.cdiv(N, BN),),
      in_specs=[
          pl.BlockSpec((NC, BN, DH), lambda i: (0, i, 0)),
          pl.BlockSpec((BN, L), lambda i: (i, 0)),
          pl.BlockSpec((BN, D), lambda i: (i, 0)),
          pl.BlockSpec((D, D), lambda i: (0, 0)),
          pl.BlockSpec((D, D), lambda i: (0, 0)),
          pl.BlockSpec((1, D), lambda i: (0, 0)),
      ],
      out_specs=pl.BlockSpec((BN, D), lambda i: (i, 0)),
      out_shape=jax.ShapeDtypeStruct((N, D), jnp.float32),
  )(agg_p, cnt_p, xin, Wl, Wr, b)


@jax.jit
def kernel(x, edge_index, W1l, W1r, b1, W2l, W2r, b2):
  src = edge_index[0].reshape(E // CHUNK, CHUNK)
  dst = edge_index[1].reshape(E // CHUNK, CHUNK)
  agg1, cnt_p = _seg_sum_cnt(x.reshape(2 * N, DH), src, dst)
  h = _dense(agg1, cnt_p, x, W1l, W1r, b1.reshape(1, D), relu=True)
  agg2 = _seg_sum(h.reshape(2 * N, DH), src, dst)
  return _dense(agg2, cnt_p, h, W2l, W2r, b2.reshape(1, D), relu=False)


# speedup vs baseline: 1.4082x; 1.4082x over previous
import jax
import jax.numpy as jnp
from jax import lax
from jax.experimental import pallas as pl
from jax.experimental.pallas import tpu as pltpu
from jax.experimental.pallas import tpu_sc as plsc

N = 10000
E = 320000
D = 128
DH = D // 2

NC = 2
NS = 16
L = 16

N_PAD = 10240
EPT = E // NS
CHUNK = 80
CPT = EPT // CHUNK
CPW = E // (NC * NS) // CHUNK
RPT = N_PAD // NS
ZROWS = 80
K = 5

_MESH = dict(core_axis_name="c", subcore_axis_name="s",
             num_cores=NC, num_subcores=NS)
_PARAMS = pltpu.CompilerParams(use_tc_tiling_on_sc=False,
                               skip_device_barrier=True)


def _seg_sum_body(x2_hbm, src_hbm, dst_hbm, agg_out, idx_all, dst_all, rows,
                  zbuf, agg_sh, sem_g, sem_s):
  cid = lax.axis_index("c")
  sid = lax.axis_index("s")

  zero16 = jnp.zeros((L,), jnp.float32)

  pltpu.sync_copy(src_hbm.at[pl.ds(sid * CPT, CPT)], idx_all)
  pltpu.sync_copy(dst_hbm.at[pl.ds(sid * CPT, CPT)], dst_all)

  def ix(i, _):
    r = i // (CHUNK // L)
    k = i % (CHUNK // L)
    sv = idx_all[r, pl.ds(k * L, L)]
    idx_all[r, pl.ds(k * L, L)] = sv * 2 + cid
    return 0
  lax.fori_loop(0, CPT * (CHUNK // L), ix, 0)

  def zb(i, _):
    r = i // (DH // L)
    k = i % (DH // L)
    zbuf[r, pl.ds(k * L, L)] = zero16
    return 0
  lax.fori_loop(0, ZROWS * (DH // L), zb, 0)

  def zcpy(j, _):
    pltpu.sync_copy(zbuf, agg_sh.at[pl.ds(sid * RPT + j * ZROWS, ZROWS)])
    return 0
  lax.fori_loop(0, RPT // ZROWS, zcpy, 0)

  plsc.subcore_barrier()

  def gather(c, k):
    return pltpu.make_async_copy(x2_hbm.at[idx_all.at[c]], rows[k], sem_g[k])

  def scatter(c, k):
    return pltpu.make_async_copy(rows[k], agg_sh.at[dst_all.at[c]], sem_s[k])

  for k in range(K - 1):
    gather(k, k).start()

  def step(q, _):
    for k in range(K):
      c = K * q + k
      gather(c, k).wait()
      pltpu.async_copy(rows[k], agg_sh.at[dst_all.at[c]], sem_s[k], add=True)
      k3 = (k + K - 1) % K

      @pl.when(c + K - 1 < CPT)
      def _():
        @pl.when(c >= 1)
        def _():
          scatter(c - 1, k3).wait()
        gather(c + K - 1, k3).start()
    return 0

  lax.fori_loop(0, CPT // K, step, 0)

  for k in range(K):
    scatter(CPT - K + k, k).wait()
  plsc.subcore_barrier()

  pltpu.sync_copy(agg_sh.at[pl.ds(sid * RPT, RPT)],
                  agg_out.at[cid, pl.ds(sid * RPT, RPT)])


_seg_sum = pl.kernel(
    _seg_sum_body,
    out_type=jax.ShapeDtypeStruct((NC, N_PAD, DH), jnp.float32),
    mesh=plsc.VectorSubcoreMesh(**_MESH),
    scratch_types=(
        pltpu.VMEM((CPT, CHUNK), jnp.int32),
        pltpu.VMEM((CPT, CHUNK), jnp.int32),
        [pltpu.VMEM((CHUNK, DH), jnp.float32) for _ in range(K)],
        pltpu.VMEM((ZROWS, DH), jnp.float32),
        pltpu.VMEM_SHARED((N_PAD, DH), jnp.float32),
        [pltpu.SemaphoreType.DMA for _ in range(K)],
        [pltpu.SemaphoreType.DMA for _ in range(K)],
    ),
    compiler_params=_PARAMS)


def _counts_body(dst_hbm, cnt_out, dst_all, ones_v, zcnt, cnt_sh, sem_c):
  cid = lax.axis_index("c")
  sid = lax.axis_index("s")
  wid = sid * NC + cid

  zero16 = jnp.zeros((L,), jnp.float32)
  ones16 = jnp.ones((L,), jnp.float32)

  pltpu.sync_copy(dst_hbm.at[pl.ds(wid * CPW, CPW)], dst_all)

  def fo(i, _):
    ones_v[i, pl.ds(0, L)] = ones16
    return 0
  lax.fori_loop(0, CHUNK, fo, 0)

  def zc(i, _):
    zcnt[i, pl.ds(0, L)] = zero16
    return 0
  lax.fori_loop(0, RPT, zc, 0)
  pltpu.sync_copy(zcnt, cnt_sh.at[pl.ds(sid * RPT, RPT)])

  plsc.subcore_barrier()

  def count(c, k):
    return pltpu.make_async_copy(ones_v, cnt_sh.at[dst_all.at[c]], sem_c[k])

  def step(q, _):
    for k in range(K):
      c = K * q + k

      @pl.when(c >= K)
      def _():
        count(c - K, k).wait()
      pltpu.async_copy(ones_v, cnt_sh.at[dst_all.at[c]], sem_c[k], add=True)
    return 0

  lax.fori_loop(0, CPW // K, step, 0)
  for k in range(K):
    count(CPW - K + k, k).wait()
  plsc.subcore_barrier()

  pltpu.sync_copy(cnt_sh.at[pl.ds(sid * RPT, RPT)],
                  cnt_out.at[cid, pl.ds(sid * RPT, RPT)])


_counts = pl.kernel(
    _counts_body,
    out_type=jax.ShapeDtypeStruct((NC, N_PAD, L), jnp.float32),
    mesh=plsc.VectorSubcoreMesh(**_MESH),
    scratch_types=(
        pltpu.VMEM((CPW, CHUNK), jnp.int32),
        pltpu.VMEM((CHUNK, L), jnp.float32),
        pltpu.VMEM((RPT, L), jnp.float32),
        pltpu.VMEM_SHARED((N_PAD, L), jnp.float32),
        [pltpu.SemaphoreType.DMA for _ in range(K)],
    ),
    compiler_params=_PARAMS)


def _dense(agg_p, cnt_p, xin, Wl, Wr, b, relu):
  BN = 512

  def body(aggp_ref, cnt_ref, x_ref, wl_ref, wr_ref, b_ref, o_ref):
    agg = jnp.concatenate([aggp_ref[0], aggp_ref[1]], axis=-1)
    cnt = cnt_ref[0, :, 0] + cnt_ref[1, :, 0]
    inv = 1.0 / jnp.maximum(cnt, 1.0)
    mean = agg * inv[:, None]
    out = (jnp.dot(mean, wl_ref[...], preferred_element_type=jnp.float32)
           + jnp.dot(x_ref[...], wr_ref[...], preferred_element_type=jnp.float32)
           + b_ref[...])
    if relu:
      out = jnp.maximum(out, 0.0)
    o_ref[...] = out

  return pl.pallas_call(
      body,
      grid=(pl.cdiv(N, BN),),
      in_specs=[
          pl.BlockSpec((NC, BN, DH), lambda i: (0, i, 0)),
          pl.BlockSpec((NC, BN, L), lambda i: (0, i, 0)),
          pl.BlockSpec((BN, D), lambda i: (i, 0)),
          pl.BlockSpec((D, D), lambda i: (0, 0)),
          pl.BlockSpec((D, D), lambda i: (0, 0)),
          pl.BlockSpec((1, D), lambda i: (0, 0)),
      ],
      out_specs=pl.BlockSpec((BN, D), lambda i: (i, 0)),
      out_shape=jax.ShapeDtypeStruct((N, D), jnp.float32),
  )(agg_p, cnt_p, xin, Wl, Wr, b)


@jax.jit
def kernel(x, edge_index, W1l, W1r, b1, W2l, W2r, b2):
  src = edge_index[0].reshape(E // CHUNK, CHUNK)
  dst = edge_index[1].reshape(E // CHUNK, CHUNK)
  cnt_p = _counts(dst)
  agg1 = _seg_sum(x.reshape(2 * N, DH), src, dst)
  h = _dense(agg1, cnt_p, x, W1l, W1r, b1.reshape(1, D), relu=True)
  agg2 = _seg_sum(h.reshape(2 * N, DH), src, dst)
  return _dense(agg2, cnt_p, h, W2l, W2r, b2.reshape(1, D), relu=False)


# speedup vs baseline: 1.4107x; 1.0017x over previous
import jax
import jax.numpy as jnp
from jax import lax
from jax.experimental import pallas as pl
from jax.experimental.pallas import tpu as pltpu
from jax.experimental.pallas import tpu_sc as plsc

N = 10000
E = 320000
D = 128
DH = D // 2

NC = 2
NS = 16
L = 16

N_PAD = 10240
EPT = E // NS
CHUNK = 80
CPT = EPT // CHUNK
CPW = E // (NC * NS) // CHUNK
RPT = N_PAD // NS
ZROWS = 80
K = 5

_MESH = dict(core_axis_name="c", subcore_axis_name="s",
             num_cores=NC, num_subcores=NS)
_PARAMS = pltpu.CompilerParams(use_tc_tiling_on_sc=False)


def _seg_sum_body(x2_hbm, src_hbm, dst_hbm, agg_out, idx_all, dst_all, rows,
                  zbuf, agg_sh, sem_g, sem_s):
  cid = lax.axis_index("c")
  sid = lax.axis_index("s")

  zero16 = jnp.zeros((L,), jnp.float32)

  pltpu.sync_copy(src_hbm.at[pl.ds(sid * CPT, CPT)], idx_all)
  pltpu.sync_copy(dst_hbm.at[pl.ds(sid * CPT, CPT)], dst_all)

  def ix(i, _):
    r = i // (CHUNK // L)
    k = i % (CHUNK // L)
    sv = idx_all[r, pl.ds(k * L, L)]
    idx_all[r, pl.ds(k * L, L)] = sv * 2 + cid
    return 0
  lax.fori_loop(0, CPT * (CHUNK // L), ix, 0)

  def zb(i, _):
    r = i // (DH // L)
    k = i % (DH // L)
    zbuf[r, pl.ds(k * L, L)] = zero16
    return 0
  lax.fori_loop(0, ZROWS * (DH // L), zb, 0)

  def zcpy(j, _):
    pltpu.sync_copy(zbuf, agg_sh.at[pl.ds(sid * RPT + j * ZROWS, ZROWS)])
    return 0
  lax.fori_loop(0, RPT // ZROWS, zcpy, 0)

  plsc.subcore_barrier()

  def gather(c, k):
    return pltpu.make_async_copy(x2_hbm.at[idx_all.at[c]], rows[k], sem_g[k])

  def scatter(c, k):
    return pltpu.make_async_copy(rows[k], agg_sh.at[dst_all.at[c]], sem_s[k])

  for k in range(K - 1):
    gather(k, k).start()

  def step(q, _):
    for k in range(K):
      c = K * q + k
      gather(c, k).wait()
      pltpu.async_copy(rows[k], agg_sh.at[dst_all.at[c]], sem_s[k], add=True)
      k3 = (k + K - 1) % K

      @pl.when(c + K - 1 < CPT)
      def _():
        @pl.when(c >= 1)
        def _():
          scatter(c - 1, k3).wait()
        gather(c + K - 1, k3).start()
    return 0

  lax.fori_loop(0, CPT // K, step, 0)

  for k in range(K):
    scatter(CPT - K + k, k).wait()
  plsc.subcore_barrier()

  pltpu.sync_copy(agg_sh.at[pl.ds(sid * RPT, RPT)],
                  agg_out.at[cid, pl.ds(sid * RPT, RPT)])


_seg_sum = pl.kernel(
    _seg_sum_body,
    out_type=jax.ShapeDtypeStruct((NC, N_PAD, DH), jnp.float32),
    mesh=plsc.VectorSubcoreMesh(**_MESH),
    scratch_types=(
        pltpu.VMEM((CPT, CHUNK), jnp.int32),
        pltpu.VMEM((CPT, CHUNK), jnp.int32),
        [pltpu.VMEM((CHUNK, DH), jnp.float32) for _ in range(K)],
        pltpu.VMEM((ZROWS, DH), jnp.float32),
        pltpu.VMEM_SHARED((N_PAD, DH), jnp.float32),
        [pltpu.SemaphoreType.DMA for _ in range(K)],
        [pltpu.SemaphoreType.DMA for _ in range(K)],
    ),
    compiler_params=_PARAMS)


def _counts_body(dst_hbm, cnt_out, dst_all, ones_v, zcnt, cnt_sh, sem_c):
  cid = lax.axis_index("c")
  sid = lax.axis_index("s")
  wid = sid * NC + cid

  zero16 = jnp.zeros((L,), jnp.float32)
  ones16 = jnp.ones((L,), jnp.float32)

  pltpu.sync_copy(dst_hbm.at[pl.ds(wid * CPW, CPW)], dst_all)

  def fo(i, _):
    ones_v[i, pl.ds(0, L)] = ones16
    return 0
  lax.fori_loop(0, CHUNK, fo, 0)

  def zc(i, _):
    zcnt[i, pl.ds(0, L)] = zero16
    return 0
  lax.fori_loop(0, RPT, zc, 0)
  pltpu.sync_copy(zcnt, cnt_sh.at[pl.ds(sid * RPT, RPT)])

  plsc.subcore_barrier()

  def count(c, k):
    return pltpu.make_async_copy(ones_v, cnt_sh.at[dst_all.at[c]], sem_c[k])

  def step(q, _):
    for k in range(K):
      c = K * q + k

      @pl.when(c >= K)
      def _():
        count(c - K, k).wait()
      pltpu.async_copy(ones_v, cnt_sh.at[dst_all.at[c]], sem_c[k], add=True)
    return 0

  lax.fori_loop(0, CPW // K, step, 0)
  for k in range(K):
    count(CPW - K + k, k).wait()
  plsc.subcore_barrier()

  pltpu.sync_copy(cnt_sh.at[pl.ds(sid * RPT, RPT)],
                  cnt_out.at[cid, pl.ds(sid * RPT, RPT)])


_counts = pl.kernel(
    _counts_body,
    out_type=jax.ShapeDtypeStruct((NC, N_PAD, L), jnp.float32),
    mesh=plsc.VectorSubcoreMesh(**_MESH),
    scratch_types=(
        pltpu.VMEM((CPW, CHUNK), jnp.int32),
        pltpu.VMEM((CHUNK, L), jnp.float32),
        pltpu.VMEM((RPT, L), jnp.float32),
        pltpu.VMEM_SHARED((N_PAD, L), jnp.float32),
        [pltpu.SemaphoreType.DMA for _ in range(K)],
    ),
    compiler_params=_PARAMS)


def _dense(agg_p, cnt_p, xin, Wl, Wr, b, relu):
  BN = 512

  def body(aggp_ref, cnt_ref, x_ref, wl_ref, wr_ref, b_ref, o_ref):
    agg = jnp.concatenate([aggp_ref[0], aggp_ref[1]], axis=-1)
    cnt = cnt_ref[0, :, 0] + cnt_ref[1, :, 0]
    inv = 1.0 / jnp.maximum(cnt, 1.0)
    mean = agg * inv[:, None]
    out = (jnp.dot(mean, wl_ref[...], preferred_element_type=jnp.float32)
           + jnp.dot(x_ref[...], wr_ref[...], preferred_element_type=jnp.float32)
           + b_ref[...])
    if relu:
      out = jnp.maximum(out, 0.0)
    o_ref[...] = out

  return pl.pallas_call(
      body,
      grid=(pl.cdiv(N, BN),),
      in_specs=[
          pl.BlockSpec((NC, BN, DH), lambda i: (0, i, 0)),
          pl.BlockSpec((NC, BN, L), lambda i: (0, i, 0)),
          pl.BlockSpec((BN, D), lambda i: (i, 0)),
          pl.BlockSpec((D, D), lambda i: (0, 0)),
          pl.BlockSpec((D, D), lambda i: (0, 0)),
          pl.BlockSpec((1, D), lambda i: (0, 0)),
      ],
      out_specs=pl.BlockSpec((BN, D), lambda i: (i, 0)),
      out_shape=jax.ShapeDtypeStruct((N, D), jnp.float32),
  )(agg_p, cnt_p, xin, Wl, Wr, b)


@jax.jit
def kernel(x, edge_index, W1l, W1r, b1, W2l, W2r, b2):
  src = edge_index[0].reshape(E // CHUNK, CHUNK)
  dst = edge_index[1].reshape(E // CHUNK, CHUNK)
  cnt_p = _counts(dst)
  agg1 = _seg_sum(x.reshape(2 * N, DH), src, dst)
  h = _dense(agg1, cnt_p, x, W1l, W1r, b1.reshape(1, D), relu=True)
  agg2 = _seg_sum(h.reshape(2 * N, DH), src, dst)
  return _dense(agg2, cnt_p, h, W2l, W2r, b2.reshape(1, D), relu=False)
